# baseline (speedup 1.0000x reference)
import jax
import jax.numpy as jnp
from jax import lax
from jax.experimental import pallas as pl
from jax.experimental.pallas import tpu as pltpu
from jax.experimental.pallas import tpu_sc as plsc

N_F = 26
VOCAB = 100000
E = 32
B = 16384

NC = 2
NS = 16
NW = NC * NS
PAIRS = N_F * E
PPW = PAIRS // NW
OC = 4096
NOC = B // OC


def _sc_body(idx_hbm, tab_hbm, out_hbm, idx_v, slab_v, out_v, sem):
    c = lax.axis_index("c")
    s = lax.axis_index("s")
    wid = s * NC + c

    def pair_body(i, _):
        f = i
        e = wid
        pltpu.sync_copy(idx_hbm.at[f], idx_v)

        pltpu.sync_copy(tab_hbm.at[f, e], slab_v)

        copies = []
        for ch in range(NOC):
            if ch >= 2:
                copies[ch - 2].wait()
            slot = ch % 2
            base = ch * OC

            @plsc.parallel_loop(0, OC // 16, 1, unroll=8)
            def _(j):
                iv = idx_v[pl.ds(base + j * 16, 16)]
                out_v[slot, pl.ds(j * 16, 16)] = plsc.load_gather(slab_v, [iv])

            copies.append(
                pltpu.async_copy(out_v.at[slot],
                                 out_hbm.at[f, e, pl.ds(base, OC)], sem))
        copies[NOC - 2].wait()
        copies[NOC - 1].wait()
        return 0

    lax.fori_loop(0, PPW, pair_body, 0)


@jax.jit
def kernel(inputs, tables):
    idx_t = inputs.T
    tab_t = tables.transpose(0, 2, 1)

    mesh = plsc.VectorSubcoreMesh(core_axis_name="c", subcore_axis_name="s")
    run = pl.kernel(
        _sc_body,
        out_type=jax.ShapeDtypeStruct((N_F, E, B), jnp.float32),
        mesh=mesh,
        scratch_types=[
            pltpu.VMEM((B,), jnp.int32),
            pltpu.VMEM((VOCAB,), jnp.float32),
            pltpu.VMEM((2, OC), jnp.float32),
            pltpu.SemaphoreType.DMA,
        ],
        compiler_params=pltpu.CompilerParams(
            needs_layout_passes=False,
            use_tc_tiling_on_sc=True,
        ),
    )
    return run(idx_t, tab_t).transpose(1, 2, 0)


# speedup vs baseline: 1.2284x; 1.2284x over previous
import jax
import jax.numpy as jnp
from jax import lax
from jax.experimental import pallas as pl
from jax.experimental.pallas import tpu as pltpu
from jax.experimental.pallas import tpu_sc as plsc

N_F = 26
VOCAB = 100000
E = 32
B = 16384

NC = 2
NS = 16
NW = NC * NS
PAIRS = N_F * E
PPW = PAIRS // NW
OC = 4096
NOC = B // OC


def _sc_body(idx_hbm, tab_hbm, out_hbm, idx_v, slab_v, out_v, sem, sem_s):
    c = lax.axis_index("c")
    s = lax.axis_index("s")
    wid = s * NC + c

    def pair_body(i, _):
        p = wid * PPW + i
        f = p >> 5
        e = p & (E - 1)
        slab_cp = pltpu.async_copy(tab_hbm.at[f, e], slab_v, sem_s)
        @pl.when((i == 0) | (f != ((p - 1) >> 5)))
        def _():
            pltpu.sync_copy(idx_hbm.at[f], idx_v)

        slab_cp.wait()

        copies = []
        for ch in range(NOC):
            if ch >= 2:
                copies[ch - 2].wait()
            slot = ch % 2
            base = ch * OC

            @plsc.parallel_loop(0, OC // 16, 1, unroll=16)
            def _(j):
                iv = idx_v[pl.ds(base + j * 16, 16)]
                out_v[slot, pl.ds(j * 16, 16)] = plsc.load_gather(slab_v, [iv])

            copies.append(
                pltpu.async_copy(out_v.at[slot],
                                 out_hbm.at[f, e, pl.ds(base, OC)], sem))
        copies[NOC - 2].wait()
        copies[NOC - 1].wait()
        return 0

    lax.fori_loop(0, PPW, pair_body, 0)


@jax.jit
def kernel(inputs, tables):
    idx_t = inputs.T
    tab_t = tables.transpose(0, 2, 1)

    mesh = plsc.VectorSubcoreMesh(core_axis_name="c", subcore_axis_name="s")
    run = pl.kernel(
        _sc_body,
        out_type=jax.ShapeDtypeStruct((N_F, E, B), jnp.float32),
        mesh=mesh,
        scratch_types=[
            pltpu.VMEM((B,), jnp.int32),
            pltpu.VMEM((VOCAB,), jnp.float32),
            pltpu.VMEM((2, OC), jnp.float32),
            pltpu.SemaphoreType.DMA,
            pltpu.SemaphoreType.DMA,
        ],
        compiler_params=pltpu.CompilerParams(
            needs_layout_passes=False,
            use_tc_tiling_on_sc=True,
        ),
    )
    return run(idx_t, tab_t).transpose(1, 2, 0)
